# Initial kernel scaffold; baseline (speedup 1.0000x reference)
import jax
import jax.numpy as jnp
from jax.experimental import pallas as pl


def kernel(x, edge_index, W_ext, W_msg):
    raise NotImplementedError("write your pallas kernel here")


# speedup vs baseline: 6.5181x; 6.5181x over previous
import dataclasses
import functools

import jax
import jax.numpy as jnp
from jax import lax
from jax.experimental import pallas as pl
from jax.experimental.pallas import tpu as pltpu
from jax.experimental.pallas import tpu_sc as plsc

_N = 10000
_E = 320000
_D = 128
_DH = _D // 2
_R = 0.5

_NC = 2
_NS = 16
_C = 80
_EPW = _E // _NS
_NCH = _EPW // _C
_NPW = _N // _NS


def _pre_body(x_ref, w_ref, p_ref):
    p_ref[...] = lax.dot_general(
        w_ref[...], x_ref[...], (((1,), (1,)), ((), ())),
        preferred_element_type=jnp.float32)


_pre = pl.pallas_call(
    _pre_body,
    out_shape=jax.ShapeDtypeStruct((2, _N), jnp.float32),
)


def _sc_body(xh_hbm, srcr_hbm, dstr_hbm, p1_hbm, p2_hbm, z2d_hbm, z1d_hbm,
             att_hbm, agg_hbm, asum_hbm, deg_hbm,
             idxs_v, idxd_v, p1_v, p2_v, att_v, rows_v, ones_v,
             agg_sh, nsum_sh, sem):
    c = lax.axis_index("c")
    s = lax.axis_index("s")
    w = c * _NS + s

    pltpu.sync_copy(srcr_hbm.at[s], idxs_v)
    pltpu.sync_copy(dstr_hbm.at[s], idxd_v)
    pltpu.sync_copy(p1_hbm, p1_v)
    pltpu.sync_copy(p2_hbm, p2_v)

    for g in range(_C // 16):
        ones_v[pl.ds(g * 16, 16)] = jnp.ones((16,), jnp.float32)

    pltpu.sync_copy(z2d_hbm.at[s], agg_sh.at[pl.ds(s * _NPW, _NPW)])

    @pl.when(s == 0)
    def _():
        pltpu.sync_copy(z1d_hbm, nsum_sh)

    plsc.subcore_barrier()

    @pl.loop(0, _NCH)
    def _chunk(j):
        for g in range(_C // 16):
            sl = pl.ds(g * 16, 16)
            s16 = idxs_v[j, sl]
            d16 = idxd_v[j, sl]
            z = plsc.load_gather(p1_v, [s16]) + plsc.load_gather(p2_v, [d16])
            ez = jnp.exp(-jnp.abs(z))
            att_v[j, sl] = jnp.where(z >= 0, 1.0, ez) / (1.0 + ez)

        pltpu.async_copy(xh_hbm.at[c].at[idxs_v.at[j]], rows_v, sem).wait()

        @pl.loop(0, _C)
        def _row(i):
            a16 = plsc.load_gather(
                att_v, [jnp.full((16,), j, jnp.int32),
                        jnp.full((16,), i, jnp.int32)])
            for seg in range(_DH // 16):
                sl = pl.ds(seg * 16, 16)
                rows_v[i, sl] = rows_v[i, sl] * a16

        pltpu.sync_copy(rows_v, agg_sh.at[idxd_v.at[j]], add=True)

        @pl.when(c == 0)
        def _():
            pltpu.sync_copy(att_v.at[j], nsum_sh.at[idxs_v.at[j]], add=True)

        @pl.when(c == 1)
        def _():
            pltpu.sync_copy(ones_v, nsum_sh.at[idxs_v.at[j]], add=True)

    @pl.when(c == 0)
    def _():
        pltpu.sync_copy(att_v, att_hbm.at[s])

    plsc.subcore_barrier()

    pltpu.sync_copy(agg_sh.at[pl.ds(s * _NPW, _NPW)], agg_hbm.at[w])

    @pl.when(s == 0)
    def _():
        @pl.when(c == 0)
        def _():
            pltpu.sync_copy(nsum_sh, asum_hbm)

        @pl.when(c == 1)
        def _():
            pltpu.sync_copy(nsum_sh, deg_hbm)


_sc_params = pltpu.CompilerParams()
if "needs_layout_passes" in pltpu.CompilerParams.__dataclass_fields__:
    _sc_params = dataclasses.replace(_sc_params, needs_layout_passes=False)
if "use_tc_tiling_on_sc" in pltpu.CompilerParams.__dataclass_fields__:
    _sc_params = dataclasses.replace(_sc_params, use_tc_tiling_on_sc=False)

_sc_main = pl.kernel(
    _sc_body,
    compiler_params=_sc_params,
    out_type=[
        jax.ShapeDtypeStruct((_NS, _NCH, _C), jnp.float32),
        jax.ShapeDtypeStruct((_NC * _NS, _NPW, _DH), jnp.float32),
        jax.ShapeDtypeStruct((_N,), jnp.float32),
        jax.ShapeDtypeStruct((_N,), jnp.float32),
    ],
    mesh=plsc.VectorSubcoreMesh(core_axis_name="c", subcore_axis_name="s"),
    scratch_types=[
        pltpu.VMEM((_NCH, _C), jnp.int32),
        pltpu.VMEM((_NCH, _C), jnp.int32),
        pltpu.VMEM((_N,), jnp.float32),
        pltpu.VMEM((_N,), jnp.float32),
        pltpu.VMEM((_NCH, _C), jnp.float32),
        pltpu.VMEM((_C, _DH), jnp.float32),
        pltpu.VMEM((_C,), jnp.float32),
        pltpu.VMEM_SHARED((_N, _DH), jnp.float32),
        pltpu.VMEM_SHARED((_N,), jnp.float32),
        pltpu.SemaphoreType.DMA,
    ],
)


def _post_body(agg_ref, att_ref, asum_ref, deg_ref, wmsg_ref,
               emb_ref, natt_ref, info_ref):
    wm = wmsg_ref[...]
    emb_ref[...] = (
        jnp.dot(agg_ref[0], wm[0:_DH, :], preferred_element_type=jnp.float32)
        + jnp.dot(agg_ref[1], wm[_DH:_D, :],
                  preferred_element_type=jnp.float32))
    natt_ref[...] = asum_ref[...] / jnp.maximum(deg_ref[...], 1.0)
    a = att_ref[...]
    f = (a * jnp.log(a / _R + 1e-6)
         + (1.0 - a) * jnp.log((1.0 - a) / (1.0 - _R + 1e-6) + 1e-6))
    info_ref[...] = (jnp.sum(f) / float(_E)).reshape(1, 1)


_post = pl.pallas_call(
    _post_body,
    out_shape=[
        jax.ShapeDtypeStruct((_N, _D), jnp.float32),
        jax.ShapeDtypeStruct((_N,), jnp.float32),
        jax.ShapeDtypeStruct((1, 1), jnp.float32),
    ],
)


def kernel(x, edge_index, W_ext, W_msg):
    srcr = edge_index[0].reshape(_NS, _NCH, _C)
    dstr = edge_index[1].reshape(_NS, _NCH, _C)
    xh = jnp.stack([x[:, :_DH], x[:, _DH:]])
    w2 = W_ext.reshape(2, _D)
    p = _pre(x, w2)
    z2d = jnp.zeros((_NS, _NPW, _DH), jnp.float32)
    z1d = jnp.zeros((_N,), jnp.float32)
    att3d, aggf, asum, deg = _sc_main(xh, srcr, dstr, p[0], p[1], z2d, z1d)
    emb, natt, info = _post(aggf.reshape(_NC, _N, _DH), att3d,
                            asum, deg, W_msg)
    return emb, att3d.reshape(_E, 1), natt, info.reshape(())
